# Initial kernel scaffold; baseline (speedup 1.0000x reference)
import jax
import jax.numpy as jnp
from jax.experimental import pallas as pl


def kernel(scattered_values, indices, queries, Wq, bq, Wk, bk):
    raise NotImplementedError("write your pallas kernel here")


# speedup vs baseline: 4.6811x; 4.6811x over previous
import functools

import jax
import jax.numpy as jnp
from jax import lax
from jax.experimental import pallas as pl
from jax.experimental.pallas import tpu as pltpu
from jax.experimental.pallas import tpu_sc as plsc

E = 320000
Q = 10000
D = 128
P = 128

NC = 2
NS = 16
NW = NC * NS
EPW = E // NW
CH = 80
NG = CH // 16
NCH = EPW // CH
QP = 10240
QS = QP // NW
QT = QP // NS

NEG = -3.0e38


@functools.cache
def _mesh():
    return plsc.VectorSubcoreMesh(core_axis_name="c", subcore_axis_name="s",
                                  num_cores=NC, num_subcores=NS)


_SC_PARAMS = pltpu.CompilerParams(needs_layout_passes=False)


def _wid():
    return lax.axis_index("s") * NC + lax.axis_index("c")


def _seg_reduce(i16, v16, kb_v, vb_v, is_max):
    ks, vs = plsc.sort_key_val(i16, v16)
    kb_v[...] = ks
    lane = lax.iota(jnp.int32, 16)
    for d in (1, 2, 4, 8):
        src = jnp.maximum(lane - d, 0)
        vb_v[...] = vs
        kg = plsc.load_gather(kb_v, [src])
        vg = plsc.load_gather(vb_v, [src])
        take = (lane >= d) & (kg == ks)
        if is_max:
            vs = jnp.maximum(vs, jnp.where(take, vg, NEG))
        else:
            vs = vs + jnp.where(take, vg, 0.0)
    knext = plsc.load_gather(kb_v, [jnp.minimum(lane + 1, 15)])
    last = (lane == 15) | (knext != ks)
    return ks, vs, last


def _k0_body(q_ref, wq_ref, bq_ref, wk_ref, t_ref):
    qp = jnp.dot(q_ref[...], wq_ref[...], preferred_element_type=jnp.float32)
    qp = qp + bq_ref[...]
    t = lax.dot_general(qp, wk_ref[...], (((1,), (1,)), ((), ())),
                        preferred_element_type=jnp.float32)
    t_ref[...] = t * (P ** -0.5)


def _project_t(queries, Wq, bq, Wk):
    bq2 = bq.reshape(1, P)
    return pl.pallas_call(
        _k0_body,
        grid=(25,),
        in_specs=[
            pl.BlockSpec((400, D), lambda i: (i, 0)),
            pl.BlockSpec((D, P), lambda i: (0, 0)),
            pl.BlockSpec((1, P), lambda i: (0, 0)),
            pl.BlockSpec((D, P), lambda i: (0, 0)),
        ],
        out_specs=pl.BlockSpec((400, P), lambda i: (i, 0)),
        out_shape=jax.ShapeDtypeStruct((Q, P), jnp.float32),
    )(queries, Wq, bq2, Wk)


def _k1_body(sv_hbm, idx_hbm, t_hbm, probs_hbm, segmax_hbm,
             idx_v, sv_v, t_v, probs_v, segmax_v, pbuf_v, kb_v, vb_v, sem):
    wid = _wid()
    base = wid * EPW
    pltpu.sync_copy(idx_hbm.at[pl.ds(base, EPW)], idx_v)

    zneg = jnp.full((16,), NEG, jnp.float32)

    def init(i, c):
        segmax_v[pl.ds(i * 16, 16)] = zneg
        return c
    lax.fori_loop(0, QP // 16, init, 0)

    rows = lax.iota(jnp.int32, 16)

    def chunk(j, c):
        pltpu.sync_copy(sv_hbm.at[pl.ds(base + j * CH, CH)], sv_v)
        pltpu.async_copy(t_hbm.at[idx_v.at[pl.ds(j * CH, CH)]], t_v,
                         sem).wait()

        for g in range(NG):
            for rr in range(16):
                r = g * 16 + rr
                a = sv_v[r, pl.ds(0, 16)] * t_v[r, pl.ds(0, 16)]
                for k in range(1, 8):
                    a = a + (sv_v[r, pl.ds(k * 16, 16)]
                             * t_v[r, pl.ds(k * 16, 16)])
                pbuf_v[pl.ds(rr * 16, 16)] = a
            p16 = plsc.load_gather(pbuf_v, [rows * 16])
            for cix in range(1, 16):
                p16 = p16 + plsc.load_gather(pbuf_v, [rows * 16 + cix])
            probs_v[pl.ds(j * CH + g * 16, 16)] = p16

            i16 = idx_v[pl.ds(j * CH + g * 16, 16)]
            ks, runmax, last = _seg_reduce(i16, p16, kb_v, vb_v, True)
            mold = plsc.load_gather(segmax_v, [ks])
            plsc.store_scatter(segmax_v, [ks], jnp.maximum(mold, runmax),
                               mask=last)
        return c
    lax.fori_loop(0, NCH, chunk, 0)

    pltpu.sync_copy(probs_v, probs_hbm.at[pl.ds(base, EPW)])
    pltpu.sync_copy(segmax_v, segmax_hbm.at[pl.ds(wid * QP, QP)])


def _pass1(sv, idxf, t):
    k = functools.partial(
        pl.kernel,
        mesh=_mesh(),
        compiler_params=_SC_PARAMS,
        out_type=(
            jax.ShapeDtypeStruct((E,), jnp.float32),
            jax.ShapeDtypeStruct((NW * QP,), jnp.float32),
        ),
        scratch_types=[
            pltpu.VMEM((EPW,), jnp.int32),
            pltpu.VMEM((CH, D), jnp.float32),
            pltpu.VMEM((CH, D), jnp.float32),
            pltpu.VMEM((EPW,), jnp.float32),
            pltpu.VMEM((QP,), jnp.float32),
            pltpu.VMEM((256,), jnp.float32),
            pltpu.VMEM((16,), jnp.int32),
            pltpu.VMEM((16,), jnp.float32),
            pltpu.SemaphoreType.DMA,
        ],
    )(_k1_body)
    return k(sv, idxf, t)


def _k2_body(parts_hbm, out_hbm, buf_v, acc_v):
    wid = _wid()
    off = wid * QS
    pltpu.sync_copy(parts_hbm.at[pl.ds(off, QS)], acc_v)

    def merge(p, c):
        pltpu.sync_copy(parts_hbm.at[pl.ds(p * QP + off, QS)], buf_v)
        for k in range(QS // 16):
            s = pl.ds(k * 16, 16)
            acc_v[s] = jnp.maximum(acc_v[s], buf_v[s])
        return c
    lax.fori_loop(1, NW, merge, 0)
    pltpu.sync_copy(acc_v, out_hbm.at[pl.ds(off, QS)])


def _merge_max(parts):
    k = functools.partial(
        pl.kernel,
        mesh=_mesh(),
        compiler_params=_SC_PARAMS,
        out_type=jax.ShapeDtypeStruct((QP,), jnp.float32),
        scratch_types=[
            pltpu.VMEM((QS,), jnp.float32),
            pltpu.VMEM((QS,), jnp.float32),
        ],
    )(_k2_body)
    return k(parts)


def _k3_body(probs_hbm, idx_hbm, segmax_hbm, exps_hbm, dpart_hbm,
             idx_v, probs_v, segmax_v, exps_v, denom_v, kb_v, vb_v):
    wid = _wid()
    base = wid * EPW
    pltpu.sync_copy(idx_hbm.at[pl.ds(base, EPW)], idx_v)
    pltpu.sync_copy(probs_hbm.at[pl.ds(base, EPW)], probs_v)
    pltpu.sync_copy(segmax_hbm, segmax_v)

    zero = jnp.zeros((16,), jnp.float32)

    def init(i, c):
        denom_v[pl.ds(i * 16, 16)] = zero
        return c
    lax.fori_loop(0, QP // 16, init, 0)

    def chunk(j, c):
        for g in range(NG):
            s = pl.ds(j * CH + g * 16, 16)
            i16 = idx_v[s]
            m16 = plsc.load_gather(segmax_v, [i16])
            e16 = jnp.exp(probs_v[s] - m16)
            exps_v[s] = e16
            ks, runsum, last = _seg_reduce(i16, e16, kb_v, vb_v, False)
            plsc.addupdate_scatter(denom_v, [ks], runsum, mask=last)
        return c
    lax.fori_loop(0, NCH, chunk, 0)

    pltpu.sync_copy(exps_v, exps_hbm.at[pl.ds(base, EPW)])
    pltpu.sync_copy(denom_v, dpart_hbm.at[pl.ds(wid * QP, QP)])


def _pass2(probs, idxf, segmax):
    k = functools.partial(
        pl.kernel,
        mesh=_mesh(),
        compiler_params=_SC_PARAMS,
        out_type=(
            jax.ShapeDtypeStruct((E,), jnp.float32),
            jax.ShapeDtypeStruct((NW * QP,), jnp.float32),
        ),
        scratch_types=[
            pltpu.VMEM((EPW,), jnp.int32),
            pltpu.VMEM((EPW,), jnp.float32),
            pltpu.VMEM((QP,), jnp.float32),
            pltpu.VMEM((EPW,), jnp.float32),
            pltpu.VMEM((QP,), jnp.float32),
            pltpu.VMEM((16,), jnp.int32),
            pltpu.VMEM((16,), jnp.float32),
        ],
    )(_k3_body)
    return k(probs, idxf, segmax)


def _k4_body(parts_hbm, out_hbm, buf_v, acc_v):
    wid = _wid()
    off = wid * QS
    pltpu.sync_copy(parts_hbm.at[pl.ds(off, QS)], acc_v)

    def merge(p, c):
        pltpu.sync_copy(parts_hbm.at[pl.ds(p * QP + off, QS)], buf_v)
        for k in range(QS // 16):
            s = pl.ds(k * 16, 16)
            acc_v[s] = acc_v[s] + buf_v[s]
        return c
    lax.fori_loop(1, NW, merge, 0)

    one = jnp.ones((16,), jnp.float32)
    for k in range(QS // 16):
        s = pl.ds(k * 16, 16)
        acc_v[s] = one / acc_v[s]
    pltpu.sync_copy(acc_v, out_hbm.at[pl.ds(off, QS)])


def _merge_rdenom(parts):
    k = functools.partial(
        pl.kernel,
        mesh=_mesh(),
        compiler_params=_SC_PARAMS,
        out_type=jax.ShapeDtypeStruct((QP,), jnp.float32),
        scratch_types=[
            pltpu.VMEM((QS,), jnp.float32),
            pltpu.VMEM((QS,), jnp.float32),
        ],
    )(_k4_body)
    return k(parts)


def _k5_body(sv_hbm, idx_hbm, exps_hbm, rden_hbm, scores_hbm, partial_hbm,
             idx_v, sv_v, w_v, e_v, rden_v, sc_v, accum_s):
    cid = lax.axis_index("c")
    sid = lax.axis_index("s")
    wid = sid * NC + cid
    base = wid * EPW
    pltpu.sync_copy(idx_hbm.at[pl.ds(base, EPW)], idx_v)
    pltpu.sync_copy(rden_hbm, rden_v)

    zero = jnp.zeros((16,), jnp.float32)

    def zrow(r, c):
        for k in range(8):
            w_v[r, pl.ds(k * 16, 16)] = zero
        return c
    lax.fori_loop(0, CH, zrow, 0)
    for z in range(QT // CH):
        pltpu.sync_copy(w_v, accum_s.at[pl.ds(sid * QT + z * CH, CH), :])
    plsc.subcore_barrier()

    def chunk(j, c):
        pltpu.sync_copy(sv_hbm.at[pl.ds(base + j * CH, CH)], sv_v)
        pltpu.sync_copy(exps_hbm.at[pl.ds(base + j * CH, CH)], e_v)

        for g in range(NG):
            s = pl.ds(g * 16, 16)
            i16 = idx_v[pl.ds(j * CH + g * 16, 16)]
            r16 = plsc.load_gather(rden_v, [i16])
            sc_v[s] = e_v[s] * r16

        def row(r, cc):
            splat = plsc.load_gather(sc_v, [jnp.full((16,), r, jnp.int32)])
            for k in range(8):
                sl = pl.ds(k * 16, 16)
                w_v[r, sl] = sv_v[r, sl] * splat
            return cc
        lax.fori_loop(0, CH, row, 0)
        pltpu.sync_copy(sc_v, scores_hbm.at[pl.ds(base + j * CH, CH)])
        pltpu.sync_copy(w_v, accum_s.at[idx_v.at[pl.ds(j * CH, CH)]],
                        add=True)
        return c
    lax.fori_loop(0, NCH, chunk, 0)

    plsc.subcore_barrier()
    pltpu.sync_copy(accum_s.at[pl.ds(sid * QT, QT), :],
                    partial_hbm.at[cid, pl.ds(sid * QT, QT), :])


def _pass3(sv, idxf, exps, rdenom):
    k = functools.partial(
        pl.kernel,
        mesh=_mesh(),
        compiler_params=_SC_PARAMS,
        out_type=(
            jax.ShapeDtypeStruct((E,), jnp.float32),
            jax.ShapeDtypeStruct((NC, QP, D), jnp.float32),
        ),
        scratch_types=[
            pltpu.VMEM((EPW,), jnp.int32),
            pltpu.VMEM((CH, D), jnp.float32),
            pltpu.VMEM((CH, D), jnp.float32),
            pltpu.VMEM((CH,), jnp.float32),
            pltpu.VMEM((QP,), jnp.float32),
            pltpu.VMEM((CH,), jnp.float32),
            pltpu.VMEM_SHARED((QP, D), jnp.float32),
        ],
    )(_k5_body)
    return k(sv, idxf, exps, rdenom)


def _k6_body(p_ref, o_ref):
    o_ref[...] = p_ref[0] + p_ref[1]


def _merge_partials(partial):
    return pl.pallas_call(
        _k6_body,
        grid=(25,),
        in_specs=[pl.BlockSpec((2, 400, D), lambda i: (0, i, 0))],
        out_specs=pl.BlockSpec((400, D), lambda i: (i, 0)),
        out_shape=jax.ShapeDtypeStruct((Q, D), jnp.float32),
    )(partial)


def kernel(scattered_values, indices, queries, Wq, bq, Wk, bk):
    del bk
    sv = scattered_values
    idxf = indices.astype(jnp.int32)
    t = _project_t(queries, Wq, bq, Wk)
    probs, segmax_part = _pass1(sv, idxf, t)
    segmax = _merge_max(segmax_part)
    exps, denom_part = _pass2(probs, idxf, segmax)
    rdenom = _merge_rdenom(denom_part)
    scores, partial = _pass3(sv, idxf, exps, rdenom)
    attn = _merge_partials(partial)
    return scores, attn


# speedup vs baseline: 6.3740x; 1.3616x over previous
import functools

import jax
import jax.numpy as jnp
from jax import lax
from jax.experimental import pallas as pl
from jax.experimental.pallas import tpu as pltpu
from jax.experimental.pallas import tpu_sc as plsc

E = 320000
Q = 10000
D = 128
P = 128

NC = 2
NS = 16
NW = NC * NS
EPW = E // NW
CH = 80
NG = CH // 16
NCH = EPW // CH
QP = 10240
QS = QP // NW
QT = QP // NS

NEG = -3.0e38


@functools.cache
def _mesh():
    return plsc.VectorSubcoreMesh(core_axis_name="c", subcore_axis_name="s",
                                  num_cores=NC, num_subcores=NS)


_SC_PARAMS = pltpu.CompilerParams(needs_layout_passes=False)


def _wid():
    return lax.axis_index("s") * NC + lax.axis_index("c")


def _seg_reduce(i16, v16, kb_v, vb_v, is_max):
    ks, vs = plsc.sort_key_val(i16, v16)
    kb_v[...] = ks
    lane = lax.iota(jnp.int32, 16)
    for d in (1, 2, 4, 8):
        src = jnp.maximum(lane - d, 0)
        vb_v[...] = vs
        kg = plsc.load_gather(kb_v, [src])
        vg = plsc.load_gather(vb_v, [src])
        take = (lane >= d) & (kg == ks)
        if is_max:
            vs = jnp.maximum(vs, jnp.where(take, vg, NEG))
        else:
            vs = vs + jnp.where(take, vg, 0.0)
    knext = plsc.load_gather(kb_v, [jnp.minimum(lane + 1, 15)])
    last = (lane == 15) | (knext != ks)
    return ks, vs, last


def _k0_body(q_ref, wq_ref, bq_ref, wk_ref, t_ref):
    qp = jnp.dot(q_ref[...], wq_ref[...], preferred_element_type=jnp.float32)
    qp = qp + bq_ref[...]
    t = lax.dot_general(qp, wk_ref[...], (((1,), (1,)), ((), ())),
                        preferred_element_type=jnp.float32)
    t_ref[...] = t * (P ** -0.5)


def _project_t(queries, Wq, bq, Wk):
    bq2 = bq.reshape(1, P)
    return pl.pallas_call(
        _k0_body,
        grid=(25,),
        in_specs=[
            pl.BlockSpec((400, D), lambda i: (i, 0)),
            pl.BlockSpec((D, P), lambda i: (0, 0)),
            pl.BlockSpec((1, P), lambda i: (0, 0)),
            pl.BlockSpec((D, P), lambda i: (0, 0)),
        ],
        out_specs=pl.BlockSpec((400, P), lambda i: (i, 0)),
        out_shape=jax.ShapeDtypeStruct((Q, P), jnp.float32),
    )(queries, Wq, bq2, Wk)


def _k1_body(sv_hbm, idx_hbm, t_hbm, probs_hbm, segmax_hbm,
             idx_v, sva_v, ta_v, svb_v, tb_v, probs_v, segmax_v, pbuf_v,
             kb_v, vb_v, sa_sv, sa_t, sb_sv, sb_t):
    wid = _wid()
    base = wid * EPW
    pltpu.sync_copy(idx_hbm.at[pl.ds(base, EPW)], idx_v)

    zneg = jnp.full((16,), NEG, jnp.float32)

    def init(i, c):
        segmax_v[pl.ds(i * 16, 16)] = zneg
        return c
    lax.fori_loop(0, QP // 16, init, 0)

    rows = lax.iota(jnp.int32, 16)

    def start(j, sv_v, t_v, s_sv, s_t):
        pltpu.make_async_copy(sv_hbm.at[pl.ds(base + j * CH, CH)], sv_v,
                              s_sv).start()
        pltpu.make_async_copy(t_hbm.at[idx_v.at[pl.ds(j * CH, CH)]], t_v,
                              s_t).start()

    def wait(j, sv_v, t_v, s_sv, s_t):
        pltpu.make_async_copy(sv_hbm.at[pl.ds(base + j * CH, CH)], sv_v,
                              s_sv).wait()
        pltpu.make_async_copy(t_hbm.at[idx_v.at[pl.ds(j * CH, CH)]], t_v,
                              s_t).wait()

    def compute(j, sv_v, t_v):
        for g in range(NG):
            for rr in range(16):
                r = g * 16 + rr
                a = sv_v[r, pl.ds(0, 16)] * t_v[r, pl.ds(0, 16)]
                for k in range(1, 8):
                    a = a + (sv_v[r, pl.ds(k * 16, 16)]
                             * t_v[r, pl.ds(k * 16, 16)])
                pbuf_v[pl.ds(rr * 16, 16)] = a
            p16 = plsc.load_gather(pbuf_v, [rows * 16])
            for cix in range(1, 16):
                p16 = p16 + plsc.load_gather(pbuf_v, [rows * 16 + cix])
            probs_v[pl.ds(j * CH + g * 16, 16)] = p16

            i16 = idx_v[pl.ds(j * CH + g * 16, 16)]
            ks, runmax, last = _seg_reduce(i16, p16, kb_v, vb_v, True)
            mold = plsc.load_gather(segmax_v, [ks])
            plsc.store_scatter(segmax_v, [ks], jnp.maximum(mold, runmax),
                               mask=last)

    start(0, sva_v, ta_v, sa_sv, sa_t)

    def pair(i, c):
        j = 2 * i
        start(j + 1, svb_v, tb_v, sb_sv, sb_t)
        wait(j, sva_v, ta_v, sa_sv, sa_t)
        compute(j, sva_v, ta_v)
        start(j + 2, sva_v, ta_v, sa_sv, sa_t)
        wait(j + 1, svb_v, tb_v, sb_sv, sb_t)
        compute(j + 1, svb_v, tb_v)
        return c
    lax.fori_loop(0, (NCH - 1) // 2, pair, 0)
    wait(NCH - 1, sva_v, ta_v, sa_sv, sa_t)
    compute(NCH - 1, sva_v, ta_v)

    pltpu.sync_copy(probs_v, probs_hbm.at[pl.ds(base, EPW)])
    pltpu.sync_copy(segmax_v, segmax_hbm.at[pl.ds(wid * QP, QP)])


def _pass1(sv, idxf, t):
    k = functools.partial(
        pl.kernel,
        mesh=_mesh(),
        compiler_params=_SC_PARAMS,
        out_type=(
            jax.ShapeDtypeStruct((E,), jnp.float32),
            jax.ShapeDtypeStruct((NW * QP,), jnp.float32),
        ),
        scratch_types=[
            pltpu.VMEM((EPW,), jnp.int32),
            pltpu.VMEM((CH, D), jnp.float32),
            pltpu.VMEM((CH, D), jnp.float32),
            pltpu.VMEM((CH, D), jnp.float32),
            pltpu.VMEM((CH, D), jnp.float32),
            pltpu.VMEM((EPW,), jnp.float32),
            pltpu.VMEM((QP,), jnp.float32),
            pltpu.VMEM((256,), jnp.float32),
            pltpu.VMEM((16,), jnp.int32),
            pltpu.VMEM((16,), jnp.float32),
            pltpu.SemaphoreType.DMA,
            pltpu.SemaphoreType.DMA,
            pltpu.SemaphoreType.DMA,
            pltpu.SemaphoreType.DMA,
        ],
    )(_k1_body)
    return k(sv, idxf, t)


def _k2_body(parts_hbm, out_hbm, buf_v, acc_v):
    wid = _wid()
    off = wid * QS
    pltpu.sync_copy(parts_hbm.at[pl.ds(off, QS)], acc_v)

    def merge(p, c):
        pltpu.sync_copy(parts_hbm.at[pl.ds(p * QP + off, QS)], buf_v)
        for k in range(QS // 16):
            s = pl.ds(k * 16, 16)
            acc_v[s] = jnp.maximum(acc_v[s], buf_v[s])
        return c
    lax.fori_loop(1, NW, merge, 0)
    pltpu.sync_copy(acc_v, out_hbm.at[pl.ds(off, QS)])


def _merge_max(parts):
    k = functools.partial(
        pl.kernel,
        mesh=_mesh(),
        compiler_params=_SC_PARAMS,
        out_type=jax.ShapeDtypeStruct((QP,), jnp.float32),
        scratch_types=[
            pltpu.VMEM((QS,), jnp.float32),
            pltpu.VMEM((QS,), jnp.float32),
        ],
    )(_k2_body)
    return k(parts)


def _k3_body(probs_hbm, idx_hbm, segmax_hbm, exps_hbm, dpart_hbm,
             idx_v, probs_v, segmax_v, exps_v, denom_v, kb_v, vb_v):
    wid = _wid()
    base = wid * EPW
    pltpu.sync_copy(idx_hbm.at[pl.ds(base, EPW)], idx_v)
    pltpu.sync_copy(probs_hbm.at[pl.ds(base, EPW)], probs_v)
    pltpu.sync_copy(segmax_hbm, segmax_v)

    zero = jnp.zeros((16,), jnp.float32)

    def init(i, c):
        denom_v[pl.ds(i * 16, 16)] = zero
        return c
    lax.fori_loop(0, QP // 16, init, 0)

    def chunk(j, c):
        for g in range(NG):
            s = pl.ds(j * CH + g * 16, 16)
            i16 = idx_v[s]
            m16 = plsc.load_gather(segmax_v, [i16])
            e16 = jnp.exp(probs_v[s] - m16)
            exps_v[s] = e16
            ks, runsum, last = _seg_reduce(i16, e16, kb_v, vb_v, False)
            plsc.addupdate_scatter(denom_v, [ks], runsum, mask=last)
        return c
    lax.fori_loop(0, NCH, chunk, 0)

    pltpu.sync_copy(exps_v, exps_hbm.at[pl.ds(base, EPW)])
    pltpu.sync_copy(denom_v, dpart_hbm.at[pl.ds(wid * QP, QP)])


def _pass2(probs, idxf, segmax):
    k = functools.partial(
        pl.kernel,
        mesh=_mesh(),
        compiler_params=_SC_PARAMS,
        out_type=(
            jax.ShapeDtypeStruct((E,), jnp.float32),
            jax.ShapeDtypeStruct((NW * QP,), jnp.float32),
        ),
        scratch_types=[
            pltpu.VMEM((EPW,), jnp.int32),
            pltpu.VMEM((EPW,), jnp.float32),
            pltpu.VMEM((QP,), jnp.float32),
            pltpu.VMEM((EPW,), jnp.float32),
            pltpu.VMEM((QP,), jnp.float32),
            pltpu.VMEM((16,), jnp.int32),
            pltpu.VMEM((16,), jnp.float32),
        ],
    )(_k3_body)
    return k(probs, idxf, segmax)


def _k4_body(parts_hbm, out_hbm, buf_v, acc_v):
    wid = _wid()
    off = wid * QS
    pltpu.sync_copy(parts_hbm.at[pl.ds(off, QS)], acc_v)

    def merge(p, c):
        pltpu.sync_copy(parts_hbm.at[pl.ds(p * QP + off, QS)], buf_v)
        for k in range(QS // 16):
            s = pl.ds(k * 16, 16)
            acc_v[s] = acc_v[s] + buf_v[s]
        return c
    lax.fori_loop(1, NW, merge, 0)

    one = jnp.ones((16,), jnp.float32)
    for k in range(QS // 16):
        s = pl.ds(k * 16, 16)
        acc_v[s] = one / acc_v[s]
    pltpu.sync_copy(acc_v, out_hbm.at[pl.ds(off, QS)])


def _merge_rdenom(parts):
    k = functools.partial(
        pl.kernel,
        mesh=_mesh(),
        compiler_params=_SC_PARAMS,
        out_type=jax.ShapeDtypeStruct((QP,), jnp.float32),
        scratch_types=[
            pltpu.VMEM((QS,), jnp.float32),
            pltpu.VMEM((QS,), jnp.float32),
        ],
    )(_k4_body)
    return k(parts)


def _k5_body(sv_hbm, idx_hbm, exps_hbm, rden_hbm, scores_hbm, partial_hbm,
             sva_v, ea_v, ia_v, sca_v, svb_v, eb_v, ib_v, scb_v,
             w_v, rden_v, accum_s,
             sa_sv, sa_e, sa_i, sb_sv, sb_e, sb_i):
    cid = lax.axis_index("c")
    sid = lax.axis_index("s")
    wid = sid * NC + cid
    base = wid * EPW
    pltpu.sync_copy(rden_hbm, rden_v)

    zero = jnp.zeros((16,), jnp.float32)

    def zrow(r, c):
        for k in range(8):
            w_v[r, pl.ds(k * 16, 16)] = zero
        return c
    lax.fori_loop(0, CH, zrow, 0)
    for z in range(QT // CH):
        pltpu.sync_copy(w_v, accum_s.at[pl.ds(sid * QT + z * CH, CH), :])
    plsc.subcore_barrier()

    def start(j, sv_v, e_v, i_v, s_sv, s_e, s_i):
        cs = pl.ds(base + j * CH, CH)
        pltpu.make_async_copy(sv_hbm.at[cs], sv_v, s_sv).start()
        pltpu.make_async_copy(exps_hbm.at[cs], e_v, s_e).start()
        pltpu.make_async_copy(idx_hbm.at[cs], i_v, s_i).start()

    def wait(j, sv_v, e_v, i_v, s_sv, s_e, s_i):
        cs = pl.ds(base + j * CH, CH)
        pltpu.make_async_copy(sv_hbm.at[cs], sv_v, s_sv).wait()
        pltpu.make_async_copy(exps_hbm.at[cs], e_v, s_e).wait()
        pltpu.make_async_copy(idx_hbm.at[cs], i_v, s_i).wait()

    def compute(j, sv_v, e_v, i_v, sc_v):
        for g in range(NG):
            s = pl.ds(g * 16, 16)
            i16 = i_v[s]
            r16 = plsc.load_gather(rden_v, [i16])
            sc_v[s] = e_v[s] * r16

        def row(r, cc):
            splat = plsc.load_gather(sc_v, [jnp.full((16,), r, jnp.int32)])
            for k in range(8):
                sl = pl.ds(k * 16, 16)
                w_v[r, sl] = sv_v[r, sl] * splat
            return cc
        lax.fori_loop(0, CH, row, 0)
        pltpu.sync_copy(sc_v, scores_hbm.at[pl.ds(base + j * CH, CH)])
        pltpu.sync_copy(w_v, accum_s.at[i_v], add=True)

    start(0, sva_v, ea_v, ia_v, sa_sv, sa_e, sa_i)

    def pair(i, c):
        j = 2 * i
        start(j + 1, svb_v, eb_v, ib_v, sb_sv, sb_e, sb_i)
        wait(j, sva_v, ea_v, ia_v, sa_sv, sa_e, sa_i)
        compute(j, sva_v, ea_v, ia_v, sca_v)
        start(j + 2, sva_v, ea_v, ia_v, sa_sv, sa_e, sa_i)
        wait(j + 1, svb_v, eb_v, ib_v, sb_sv, sb_e, sb_i)
        compute(j + 1, svb_v, eb_v, ib_v, scb_v)
        return c
    lax.fori_loop(0, (NCH - 1) // 2, pair, 0)
    wait(NCH - 1, sva_v, ea_v, ia_v, sa_sv, sa_e, sa_i)
    compute(NCH - 1, sva_v, ea_v, ia_v, sca_v)

    plsc.subcore_barrier()
    pltpu.sync_copy(accum_s.at[pl.ds(sid * QT, QT), :],
                    partial_hbm.at[cid, pl.ds(sid * QT, QT), :])


def _pass3(sv, idxf, exps, rdenom):
    k = functools.partial(
        pl.kernel,
        mesh=_mesh(),
        compiler_params=_SC_PARAMS,
        out_type=(
            jax.ShapeDtypeStruct((E,), jnp.float32),
            jax.ShapeDtypeStruct((NC, QP, D), jnp.float32),
        ),
        scratch_types=[
            pltpu.VMEM((CH, D), jnp.float32),
            pltpu.VMEM((CH,), jnp.float32),
            pltpu.VMEM((CH,), jnp.int32),
            pltpu.VMEM((CH,), jnp.float32),
            pltpu.VMEM((CH, D), jnp.float32),
            pltpu.VMEM((CH,), jnp.float32),
            pltpu.VMEM((CH,), jnp.int32),
            pltpu.VMEM((CH,), jnp.float32),
            pltpu.VMEM((CH, D), jnp.float32),
            pltpu.VMEM((QP,), jnp.float32),
            pltpu.VMEM_SHARED((QP, D), jnp.float32),
            pltpu.SemaphoreType.DMA,
            pltpu.SemaphoreType.DMA,
            pltpu.SemaphoreType.DMA,
            pltpu.SemaphoreType.DMA,
            pltpu.SemaphoreType.DMA,
            pltpu.SemaphoreType.DMA,
        ],
    )(_k5_body)
    return k(sv, idxf, exps, rdenom)


def _k6_body(p_ref, o_ref):
    o_ref[...] = p_ref[0] + p_ref[1]


def _merge_partials(partial):
    return pl.pallas_call(
        _k6_body,
        grid=(25,),
        in_specs=[pl.BlockSpec((2, 400, D), lambda i: (0, i, 0))],
        out_specs=pl.BlockSpec((400, D), lambda i: (i, 0)),
        out_shape=jax.ShapeDtypeStruct((Q, D), jnp.float32),
    )(partial)


def kernel(scattered_values, indices, queries, Wq, bq, Wk, bk):
    del bk
    sv = scattered_values
    idxf = indices.astype(jnp.int32)
    t = _project_t(queries, Wq, bq, Wk)
    probs, segmax_part = _pass1(sv, idxf, t)
    segmax = _merge_max(segmax_part)
    exps, denom_part = _pass2(probs, idxf, segmax)
    rdenom = _merge_rdenom(denom_part)
    scores, partial = _pass3(sv, idxf, exps, rdenom)
    attn = _merge_partials(partial)
    return scores, attn


# speedup vs baseline: 6.5720x; 1.0311x over previous
import functools

import jax
import jax.numpy as jnp
from jax import lax
from jax.experimental import pallas as pl
from jax.experimental.pallas import tpu as pltpu
from jax.experimental.pallas import tpu_sc as plsc

E = 320000
Q = 10000
D = 128
P = 128

NC = 2
NS = 16
NW = NC * NS
EPW = E // NW
CH = 80
NG = CH // 16
NCH = EPW // CH
QP = 10240
QS = QP // NW
QT = QP // NS

NEG = -3.0e38


@functools.cache
def _mesh():
    return plsc.VectorSubcoreMesh(core_axis_name="c", subcore_axis_name="s",
                                  num_cores=NC, num_subcores=NS)


_SC_PARAMS = pltpu.CompilerParams(needs_layout_passes=False)


def _wid():
    return lax.axis_index("s") * NC + lax.axis_index("c")


def _seg_reduce(i16, v16, kb_v, vb_v, is_max):
    ks, vs = plsc.sort_key_val(i16, v16)
    kb_v[...] = ks
    lane = lax.iota(jnp.int32, 16)
    for d in (1, 2, 4, 8):
        src = jnp.maximum(lane - d, 0)
        vb_v[...] = vs
        kg = plsc.load_gather(kb_v, [src])
        vg = plsc.load_gather(vb_v, [src])
        take = (lane >= d) & (kg == ks)
        if is_max:
            vs = jnp.maximum(vs, jnp.where(take, vg, NEG))
        else:
            vs = vs + jnp.where(take, vg, 0.0)
    knext = plsc.load_gather(kb_v, [jnp.minimum(lane + 1, 15)])
    last = (lane == 15) | (knext != ks)
    return ks, vs, last


def _k0_body(q_ref, wq_ref, bq_ref, wk_ref, t_ref):
    qp = jnp.dot(q_ref[...], wq_ref[...], preferred_element_type=jnp.float32)
    qp = qp + bq_ref[...]
    t = lax.dot_general(qp, wk_ref[...], (((1,), (1,)), ((), ())),
                        preferred_element_type=jnp.float32)
    t_ref[...] = t * (P ** -0.5)


def _project_t(queries, Wq, bq, Wk):
    bq2 = bq.reshape(1, P)
    return pl.pallas_call(
        _k0_body,
        grid=(25,),
        in_specs=[
            pl.BlockSpec((400, D), lambda i: (i, 0)),
            pl.BlockSpec((D, P), lambda i: (0, 0)),
            pl.BlockSpec((1, P), lambda i: (0, 0)),
            pl.BlockSpec((D, P), lambda i: (0, 0)),
        ],
        out_specs=pl.BlockSpec((400, P), lambda i: (i, 0)),
        out_shape=jax.ShapeDtypeStruct((Q, P), jnp.float32),
    )(queries, Wq, bq2, Wk)


def _k1_body(sv_hbm, idx_hbm, t_hbm, probs_hbm, segmax_hbm,
             idx_v, sva_v, ta_v, svb_v, tb_v, probs_v, segmax_v, pbuf_v,
             kb_v, vb_v, sa_sv, sa_t, sb_sv, sb_t):
    wid = _wid()
    base = wid * EPW
    pltpu.sync_copy(idx_hbm.at[pl.ds(base, EPW)], idx_v)

    zneg = jnp.full((16,), NEG, jnp.float32)

    def init(i, c):
        segmax_v[pl.ds(i * 16, 16)] = zneg
        return c
    lax.fori_loop(0, QP // 16, init, 0)

    rows = lax.iota(jnp.int32, 16)

    def start(j, sv_v, t_v, s_sv, s_t):
        pltpu.make_async_copy(sv_hbm.at[pl.ds(base + j * CH, CH)], sv_v,
                              s_sv).start()
        pltpu.make_async_copy(t_hbm.at[idx_v.at[pl.ds(j * CH, CH)]], t_v,
                              s_t).start()

    def wait(j, sv_v, t_v, s_sv, s_t):
        pltpu.make_async_copy(sv_hbm.at[pl.ds(base + j * CH, CH)], sv_v,
                              s_sv).wait()
        pltpu.make_async_copy(t_hbm.at[idx_v.at[pl.ds(j * CH, CH)]], t_v,
                              s_t).wait()

    def compute(j, sv_v, t_v):
        for g in range(NG):
            for rr in range(16):
                r = g * 16 + rr
                a = sv_v[r, pl.ds(0, 16)] * t_v[r, pl.ds(0, 16)]
                for k in range(1, 8):
                    a = a + (sv_v[r, pl.ds(k * 16, 16)]
                             * t_v[r, pl.ds(k * 16, 16)])
                pbuf_v[pl.ds(rr * 16, 16)] = a
            p16 = plsc.load_gather(pbuf_v, [rows * 16])
            for cix in range(1, 16):
                p16 = p16 + plsc.load_gather(pbuf_v, [rows * 16 + cix])
            probs_v[pl.ds(j * CH + g * 16, 16)] = p16

            i16 = idx_v[pl.ds(j * CH + g * 16, 16)]
            ks, runmax, last = _seg_reduce(i16, p16, kb_v, vb_v, True)
            mold = plsc.load_gather(segmax_v, [ks])
            plsc.store_scatter(segmax_v, [ks], jnp.maximum(mold, runmax),
                               mask=last)

    start(0, sva_v, ta_v, sa_sv, sa_t)

    def pair(i, c):
        j = 2 * i
        start(j + 1, svb_v, tb_v, sb_sv, sb_t)
        wait(j, sva_v, ta_v, sa_sv, sa_t)
        compute(j, sva_v, ta_v)
        start(j + 2, sva_v, ta_v, sa_sv, sa_t)
        wait(j + 1, svb_v, tb_v, sb_sv, sb_t)
        compute(j + 1, svb_v, tb_v)
        return c
    lax.fori_loop(0, (NCH - 1) // 2, pair, 0)
    wait(NCH - 1, sva_v, ta_v, sa_sv, sa_t)
    compute(NCH - 1, sva_v, ta_v)

    pltpu.sync_copy(probs_v, probs_hbm.at[pl.ds(base, EPW)])
    pltpu.sync_copy(segmax_v, segmax_hbm.at[pl.ds(wid * QP, QP)])


def _pass1(sv, idxf, t):
    k = functools.partial(
        pl.kernel,
        mesh=_mesh(),
        compiler_params=_SC_PARAMS,
        out_type=(
            jax.ShapeDtypeStruct((E,), jnp.float32),
            jax.ShapeDtypeStruct((NW * QP,), jnp.float32),
        ),
        scratch_types=[
            pltpu.VMEM((EPW,), jnp.int32),
            pltpu.VMEM((CH, D), jnp.float32),
            pltpu.VMEM((CH, D), jnp.float32),
            pltpu.VMEM((CH, D), jnp.float32),
            pltpu.VMEM((CH, D), jnp.float32),
            pltpu.VMEM((EPW,), jnp.float32),
            pltpu.VMEM((QP,), jnp.float32),
            pltpu.VMEM((256,), jnp.float32),
            pltpu.VMEM((16,), jnp.int32),
            pltpu.VMEM((16,), jnp.float32),
            pltpu.SemaphoreType.DMA,
            pltpu.SemaphoreType.DMA,
            pltpu.SemaphoreType.DMA,
            pltpu.SemaphoreType.DMA,
        ],
    )(_k1_body)
    return k(sv, idxf, t)


def _k2_body(parts_hbm, out_hbm, buf_v, acc_v):
    wid = _wid()
    off = wid * QS
    pltpu.sync_copy(parts_hbm.at[pl.ds(off, QS)], acc_v)

    def merge(p, c):
        pltpu.sync_copy(parts_hbm.at[pl.ds(p * QP + off, QS)], buf_v)
        for k in range(QS // 16):
            s = pl.ds(k * 16, 16)
            acc_v[s] = jnp.maximum(acc_v[s], buf_v[s])
        return c
    lax.fori_loop(1, NW, merge, 0)
    pltpu.sync_copy(acc_v, out_hbm.at[pl.ds(off, QS)])


def _merge_max(parts):
    k = functools.partial(
        pl.kernel,
        mesh=_mesh(),
        compiler_params=_SC_PARAMS,
        out_type=jax.ShapeDtypeStruct((QP,), jnp.float32),
        scratch_types=[
            pltpu.VMEM((QS,), jnp.float32),
            pltpu.VMEM((QS,), jnp.float32),
        ],
    )(_k2_body)
    return k(parts)


def _k3_body(probs_hbm, idx_hbm, segmax_hbm, exps_hbm, dpart_hbm,
             idx_v, probs_v, segmax_v, exps_v, denom_v, kb_v, vb_v):
    wid = _wid()
    base = wid * EPW
    pltpu.sync_copy(idx_hbm.at[pl.ds(base, EPW)], idx_v)
    pltpu.sync_copy(probs_hbm.at[pl.ds(base, EPW)], probs_v)
    pltpu.sync_copy(segmax_hbm, segmax_v)

    zero = jnp.zeros((16,), jnp.float32)

    def init(i, c):
        denom_v[pl.ds(i * 16, 16)] = zero
        return c
    lax.fori_loop(0, QP // 16, init, 0)

    def chunk(j, c):
        for g in range(NG):
            s = pl.ds(j * CH + g * 16, 16)
            i16 = idx_v[s]
            m16 = plsc.load_gather(segmax_v, [i16])
            e16 = jnp.exp(probs_v[s] - m16)
            exps_v[s] = e16
            ks, runsum, last = _seg_reduce(i16, e16, kb_v, vb_v, False)
            plsc.addupdate_scatter(denom_v, [ks], runsum, mask=last)
        return c
    lax.fori_loop(0, NCH, chunk, 0)

    pltpu.sync_copy(exps_v, exps_hbm.at[pl.ds(base, EPW)])
    pltpu.sync_copy(denom_v, dpart_hbm.at[pl.ds(wid * QP, QP)])


def _pass2(probs, idxf, segmax):
    k = functools.partial(
        pl.kernel,
        mesh=_mesh(),
        compiler_params=_SC_PARAMS,
        out_type=(
            jax.ShapeDtypeStruct((E,), jnp.float32),
            jax.ShapeDtypeStruct((NW * QP,), jnp.float32),
        ),
        scratch_types=[
            pltpu.VMEM((EPW,), jnp.int32),
            pltpu.VMEM((EPW,), jnp.float32),
            pltpu.VMEM((QP,), jnp.float32),
            pltpu.VMEM((EPW,), jnp.float32),
            pltpu.VMEM((QP,), jnp.float32),
            pltpu.VMEM((16,), jnp.int32),
            pltpu.VMEM((16,), jnp.float32),
        ],
    )(_k3_body)
    return k(probs, idxf, segmax)


def _k4_body(parts_hbm, out_hbm, buf_v, acc_v):
    wid = _wid()
    off = wid * QS
    pltpu.sync_copy(parts_hbm.at[pl.ds(off, QS)], acc_v)

    def merge(p, c):
        pltpu.sync_copy(parts_hbm.at[pl.ds(p * QP + off, QS)], buf_v)
        for k in range(QS // 16):
            s = pl.ds(k * 16, 16)
            acc_v[s] = acc_v[s] + buf_v[s]
        return c
    lax.fori_loop(1, NW, merge, 0)

    one = jnp.ones((16,), jnp.float32)
    for k in range(QS // 16):
        s = pl.ds(k * 16, 16)
        acc_v[s] = one / acc_v[s]
    pltpu.sync_copy(acc_v, out_hbm.at[pl.ds(off, QS)])


def _merge_rdenom(parts):
    k = functools.partial(
        pl.kernel,
        mesh=_mesh(),
        compiler_params=_SC_PARAMS,
        out_type=jax.ShapeDtypeStruct((QP,), jnp.float32),
        scratch_types=[
            pltpu.VMEM((QS,), jnp.float32),
            pltpu.VMEM((QS,), jnp.float32),
        ],
    )(_k4_body)
    return k(parts)


def _k5_body(sv_hbm, idx_hbm, exps_hbm, rden_hbm, scores_hbm, partial_hbm,
             sva_v, ea_v, ia_v, sca_v, svb_v, eb_v, ib_v, scb_v,
             w_v, rden_v, accum_s,
             sa_sv, sa_e, sa_i, sb_sv, sb_e, sb_i, s_sc, s_w0, s_w1):
    cid = lax.axis_index("c")
    sid = lax.axis_index("s")
    wid = sid * NC + cid
    base = wid * EPW
    pltpu.sync_copy(rden_hbm, rden_v)

    zero = jnp.zeros((16,), jnp.float32)

    def zrow(r, c):
        for k in range(8):
            w_v[r, pl.ds(k * 16, 16)] = zero
        return c
    lax.fori_loop(0, CH, zrow, 0)
    for z in range(QT // CH):
        pltpu.sync_copy(w_v, accum_s.at[pl.ds(sid * QT + z * CH, CH), :])
    plsc.subcore_barrier()

    def start(j, sv_v, e_v, i_v, s_sv, s_e, s_i):
        cs = pl.ds(base + j * CH, CH)
        pltpu.make_async_copy(sv_hbm.at[cs], sv_v, s_sv).start()
        pltpu.make_async_copy(exps_hbm.at[cs], e_v, s_e).start()
        pltpu.make_async_copy(idx_hbm.at[cs], i_v, s_i).start()

    def wait(j, sv_v, e_v, i_v, s_sv, s_e, s_i):
        cs = pl.ds(base + j * CH, CH)
        pltpu.make_async_copy(sv_hbm.at[cs], sv_v, s_sv).wait()
        pltpu.make_async_copy(exps_hbm.at[cs], e_v, s_e).wait()
        pltpu.make_async_copy(idx_hbm.at[cs], i_v, s_i).wait()

    HH = CH // 2

    def compute(j, sv_v, e_v, i_v, sc_v, s_sc, s_w0, s_w1):
        for g in range(NG):
            s = pl.ds(g * 16, 16)
            i16 = i_v[s]
            r16 = plsc.load_gather(rden_v, [i16])
            sc_v[s] = e_v[s] * r16

        dsc = pltpu.async_copy(sc_v, scores_hbm.at[pl.ds(base + j * CH, CH)],
                               s_sc)

        dws = []
        for h, s_w in ((0, s_w0), (1, s_w1)):
            def row(r, cc, h=h):
                splat = plsc.load_gather(
                    sc_v, [jnp.full((16,), h * HH, jnp.int32) + r])
                for k in range(8):
                    sl = pl.ds(k * 16, 16)
                    w_v[h * HH + r, sl] = sv_v[h * HH + r, sl] * splat
                return cc
            lax.fori_loop(0, HH, row, 0)
            dws.append(pltpu.async_copy(
                w_v.at[pl.ds(h * HH, HH), :],
                accum_s.at[i_v.at[pl.ds(h * HH, HH)]], s_w, add=True))
        dsc.wait()
        for d in dws:
            d.wait()

    start(0, sva_v, ea_v, ia_v, sa_sv, sa_e, sa_i)

    def pair(i, c):
        j = 2 * i
        start(j + 1, svb_v, eb_v, ib_v, sb_sv, sb_e, sb_i)
        wait(j, sva_v, ea_v, ia_v, sa_sv, sa_e, sa_i)
        compute(j, sva_v, ea_v, ia_v, sca_v, s_sc, s_w0, s_w1)
        start(j + 2, sva_v, ea_v, ia_v, sa_sv, sa_e, sa_i)
        wait(j + 1, svb_v, eb_v, ib_v, sb_sv, sb_e, sb_i)
        compute(j + 1, svb_v, eb_v, ib_v, scb_v, s_sc, s_w0, s_w1)
        return c
    lax.fori_loop(0, (NCH - 1) // 2, pair, 0)
    wait(NCH - 1, sva_v, ea_v, ia_v, sa_sv, sa_e, sa_i)
    compute(NCH - 1, sva_v, ea_v, ia_v, sca_v, s_sc, s_w0, s_w1)

    plsc.subcore_barrier()
    pltpu.sync_copy(accum_s.at[pl.ds(sid * QT, QT), :],
                    partial_hbm.at[cid, pl.ds(sid * QT, QT), :])


def _pass3(sv, idxf, exps, rdenom):
    k = functools.partial(
        pl.kernel,
        mesh=_mesh(),
        compiler_params=_SC_PARAMS,
        out_type=(
            jax.ShapeDtypeStruct((E,), jnp.float32),
            jax.ShapeDtypeStruct((NC, QP, D), jnp.float32),
        ),
        scratch_types=[
            pltpu.VMEM((CH, D), jnp.float32),
            pltpu.VMEM((CH,), jnp.float32),
            pltpu.VMEM((CH,), jnp.int32),
            pltpu.VMEM((CH,), jnp.float32),
            pltpu.VMEM((CH, D), jnp.float32),
            pltpu.VMEM((CH,), jnp.float32),
            pltpu.VMEM((CH,), jnp.int32),
            pltpu.VMEM((CH,), jnp.float32),
            pltpu.VMEM((CH, D), jnp.float32),
            pltpu.VMEM((QP,), jnp.float32),
            pltpu.VMEM_SHARED((QP, D), jnp.float32),
            pltpu.SemaphoreType.DMA,
            pltpu.SemaphoreType.DMA,
            pltpu.SemaphoreType.DMA,
            pltpu.SemaphoreType.DMA,
            pltpu.SemaphoreType.DMA,
            pltpu.SemaphoreType.DMA,
            pltpu.SemaphoreType.DMA,
            pltpu.SemaphoreType.DMA,
            pltpu.SemaphoreType.DMA,
        ],
    )(_k5_body)
    return k(sv, idxf, exps, rdenom)


def _k6_body(p_ref, o_ref):
    o_ref[...] = p_ref[0] + p_ref[1]


def _merge_partials(partial):
    return pl.pallas_call(
        _k6_body,
        grid=(25,),
        in_specs=[pl.BlockSpec((2, 400, D), lambda i: (0, i, 0))],
        out_specs=pl.BlockSpec((400, D), lambda i: (i, 0)),
        out_shape=jax.ShapeDtypeStruct((Q, D), jnp.float32),
    )(partial)


def kernel(scattered_values, indices, queries, Wq, bq, Wk, bk):
    del bk
    sv = scattered_values
    idxf = indices.astype(jnp.int32)
    t = _project_t(queries, Wq, bq, Wk)
    probs, segmax_part = _pass1(sv, idxf, t)
    segmax = _merge_max(segmax_part)
    exps, denom_part = _pass2(probs, idxf, segmax)
    rdenom = _merge_rdenom(denom_part)
    scores, partial = _pass3(sv, idxf, exps, rdenom)
    attn = _merge_partials(partial)
    return scores, attn


# speedup vs baseline: 6.7572x; 1.0282x over previous
import functools

import jax
import jax.numpy as jnp
from jax import lax
from jax.experimental import pallas as pl
from jax.experimental.pallas import tpu as pltpu
from jax.experimental.pallas import tpu_sc as plsc

E = 320000
Q = 10000
D = 128
P = 128

NC = 2
NS = 16
NW = NC * NS
EPW = E // NW
CH = 80
NG = CH // 16
NCH = EPW // CH
QP = 10240
QS = QP // NW
QT = QP // NS

NEG = -3.0e38


@functools.cache
def _mesh():
    return plsc.VectorSubcoreMesh(core_axis_name="c", subcore_axis_name="s",
                                  num_cores=NC, num_subcores=NS)


_SC_PARAMS = pltpu.CompilerParams(needs_layout_passes=False)


def _wid():
    return lax.axis_index("s") * NC + lax.axis_index("c")


def _seg_reduce(i16, v16, is_max):
    ks, vs = plsc.sort_key_val(i16, v16)
    lane = lax.iota(jnp.int32, 16)
    for d in (1, 2, 4, 8):
        src = jnp.maximum(lane - d, 0)
        kg = ks.at[src].get(mode="promise_in_bounds")
        vg = vs.at[src].get(mode="promise_in_bounds")
        take = (lane >= d) & (kg == ks)
        if is_max:
            vs = jnp.maximum(vs, jnp.where(take, vg, NEG))
        else:
            vs = vs + jnp.where(take, vg, 0.0)
    knext = ks.at[jnp.minimum(lane + 1, 15)].get(
        mode="promise_in_bounds")
    last = (lane == 15) | (knext != ks)
    return ks, vs, last


def _k0_body(q_ref, wq_ref, bq_ref, wk_ref, t_ref):
    qp = jnp.dot(q_ref[...], wq_ref[...], preferred_element_type=jnp.float32)
    qp = qp + bq_ref[...]
    t = lax.dot_general(qp, wk_ref[...], (((1,), (1,)), ((), ())),
                        preferred_element_type=jnp.float32)
    t_ref[...] = t * (P ** -0.5)


def _project_t(queries, Wq, bq, Wk):
    bq2 = bq.reshape(1, P)
    return pl.pallas_call(
        _k0_body,
        grid=(25,),
        in_specs=[
            pl.BlockSpec((400, D), lambda i: (i, 0)),
            pl.BlockSpec((D, P), lambda i: (0, 0)),
            pl.BlockSpec((1, P), lambda i: (0, 0)),
            pl.BlockSpec((D, P), lambda i: (0, 0)),
        ],
        out_specs=pl.BlockSpec((400, P), lambda i: (i, 0)),
        out_shape=jax.ShapeDtypeStruct((Q, P), jnp.float32),
    )(queries, Wq, bq2, Wk)


def _k1_body(sv_hbm, idx_hbm, t_hbm, probs_hbm, segmax_hbm,
             idx_v, sva_v, ta_v, svb_v, tb_v, probs_v, segmax_v, pbuf_v,
             sa_sv, sa_t, sb_sv, sb_t):
    wid = _wid()
    base = wid * EPW
    pltpu.sync_copy(idx_hbm.at[pl.ds(base, EPW)], idx_v)

    zneg = jnp.full((16,), NEG, jnp.float32)

    def init(i, c):
        segmax_v[pl.ds(i * 16, 16)] = zneg
        return c
    lax.fori_loop(0, QP // 16, init, 0)

    rows = lax.iota(jnp.int32, 16)

    def start(j, sv_v, t_v, s_sv, s_t):
        pltpu.make_async_copy(sv_hbm.at[pl.ds(base + j * CH, CH)], sv_v,
                              s_sv).start()
        pltpu.make_async_copy(t_hbm.at[idx_v.at[pl.ds(j * CH, CH)]], t_v,
                              s_t).start()

    def wait(j, sv_v, t_v, s_sv, s_t):
        pltpu.make_async_copy(sv_hbm.at[pl.ds(base + j * CH, CH)], sv_v,
                              s_sv).wait()
        pltpu.make_async_copy(t_hbm.at[idx_v.at[pl.ds(j * CH, CH)]], t_v,
                              s_t).wait()

    def compute(j, sv_v, t_v):
        for g in range(NG):
            for rr in range(16):
                r = g * 16 + rr
                a = sv_v[r, pl.ds(0, 16)] * t_v[r, pl.ds(0, 16)]
                for k in range(1, 8):
                    a = a + (sv_v[r, pl.ds(k * 16, 16)]
                             * t_v[r, pl.ds(k * 16, 16)])
                pbuf_v[pl.ds(rr * 16, 16)] = a
            p16 = plsc.load_gather(pbuf_v, [rows * 16])
            for cix in range(1, 16):
                p16 = p16 + plsc.load_gather(pbuf_v, [rows * 16 + cix])
            probs_v[pl.ds(j * CH + g * 16, 16)] = p16

            i16 = idx_v[pl.ds(j * CH + g * 16, 16)]
            ks, runmax, last = _seg_reduce(i16, p16, True)
            mold = plsc.load_gather(segmax_v, [ks])
            plsc.store_scatter(segmax_v, [ks], jnp.maximum(mold, runmax),
                               mask=last)

    start(0, sva_v, ta_v, sa_sv, sa_t)

    def pair(i, c):
        j = 2 * i
        start(j + 1, svb_v, tb_v, sb_sv, sb_t)
        wait(j, sva_v, ta_v, sa_sv, sa_t)
        compute(j, sva_v, ta_v)
        start(j + 2, sva_v, ta_v, sa_sv, sa_t)
        wait(j + 1, svb_v, tb_v, sb_sv, sb_t)
        compute(j + 1, svb_v, tb_v)
        return c
    lax.fori_loop(0, (NCH - 1) // 2, pair, 0)
    wait(NCH - 1, sva_v, ta_v, sa_sv, sa_t)
    compute(NCH - 1, sva_v, ta_v)

    pltpu.sync_copy(probs_v, probs_hbm.at[pl.ds(base, EPW)])
    pltpu.sync_copy(segmax_v, segmax_hbm.at[pl.ds(wid * QP, QP)])


def _pass1(sv, idxf, t):
    k = functools.partial(
        pl.kernel,
        mesh=_mesh(),
        compiler_params=_SC_PARAMS,
        out_type=(
            jax.ShapeDtypeStruct((E,), jnp.float32),
            jax.ShapeDtypeStruct((NW * QP,), jnp.float32),
        ),
        scratch_types=[
            pltpu.VMEM((EPW,), jnp.int32),
            pltpu.VMEM((CH, D), jnp.float32),
            pltpu.VMEM((CH, D), jnp.float32),
            pltpu.VMEM((CH, D), jnp.float32),
            pltpu.VMEM((CH, D), jnp.float32),
            pltpu.VMEM((EPW,), jnp.float32),
            pltpu.VMEM((QP,), jnp.float32),
            pltpu.VMEM((256,), jnp.float32),
            pltpu.SemaphoreType.DMA,
            pltpu.SemaphoreType.DMA,
            pltpu.SemaphoreType.DMA,
            pltpu.SemaphoreType.DMA,
        ],
    )(_k1_body)
    return k(sv, idxf, t)


def _k2_body(parts_hbm, out_hbm, buf_v, acc_v):
    wid = _wid()
    off = wid * QS
    pltpu.sync_copy(parts_hbm.at[pl.ds(off, QS)], acc_v)

    def merge(p, c):
        pltpu.sync_copy(parts_hbm.at[pl.ds(p * QP + off, QS)], buf_v)
        for k in range(QS // 16):
            s = pl.ds(k * 16, 16)
            acc_v[s] = jnp.maximum(acc_v[s], buf_v[s])
        return c
    lax.fori_loop(1, NW, merge, 0)
    pltpu.sync_copy(acc_v, out_hbm.at[pl.ds(off, QS)])


def _merge_max(parts):
    k = functools.partial(
        pl.kernel,
        mesh=_mesh(),
        compiler_params=_SC_PARAMS,
        out_type=jax.ShapeDtypeStruct((QP,), jnp.float32),
        scratch_types=[
            pltpu.VMEM((QS,), jnp.float32),
            pltpu.VMEM((QS,), jnp.float32),
        ],
    )(_k2_body)
    return k(parts)


def _k3_body(probs_hbm, idx_hbm, segmax_hbm, exps_hbm, dpart_hbm,
             idx_v, probs_v, segmax_v, exps_v, denom_v):
    wid = _wid()
    base = wid * EPW
    pltpu.sync_copy(idx_hbm.at[pl.ds(base, EPW)], idx_v)
    pltpu.sync_copy(probs_hbm.at[pl.ds(base, EPW)], probs_v)
    pltpu.sync_copy(segmax_hbm, segmax_v)

    zero = jnp.zeros((16,), jnp.float32)

    def init(i, c):
        denom_v[pl.ds(i * 16, 16)] = zero
        return c
    lax.fori_loop(0, QP // 16, init, 0)

    def chunk(j, c):
        for g in range(NG):
            s = pl.ds(j * CH + g * 16, 16)
            i16 = idx_v[s]
            m16 = plsc.load_gather(segmax_v, [i16])
            e16 = jnp.exp(probs_v[s] - m16)
            exps_v[s] = e16
            ks, runsum, last = _seg_reduce(i16, e16, False)
            plsc.addupdate_scatter(denom_v, [ks], runsum, mask=last)
        return c
    lax.fori_loop(0, NCH, chunk, 0)

    pltpu.sync_copy(exps_v, exps_hbm.at[pl.ds(base, EPW)])
    pltpu.sync_copy(denom_v, dpart_hbm.at[pl.ds(wid * QP, QP)])


def _pass2(probs, idxf, segmax):
    k = functools.partial(
        pl.kernel,
        mesh=_mesh(),
        compiler_params=_SC_PARAMS,
        out_type=(
            jax.ShapeDtypeStruct((E,), jnp.float32),
            jax.ShapeDtypeStruct((NW * QP,), jnp.float32),
        ),
        scratch_types=[
            pltpu.VMEM((EPW,), jnp.int32),
            pltpu.VMEM((EPW,), jnp.float32),
            pltpu.VMEM((QP,), jnp.float32),
            pltpu.VMEM((EPW,), jnp.float32),
            pltpu.VMEM((QP,), jnp.float32),
        ],
    )(_k3_body)
    return k(probs, idxf, segmax)


def _k4_body(parts_hbm, out_hbm, buf_v, acc_v):
    wid = _wid()
    off = wid * QS
    pltpu.sync_copy(parts_hbm.at[pl.ds(off, QS)], acc_v)

    def merge(p, c):
        pltpu.sync_copy(parts_hbm.at[pl.ds(p * QP + off, QS)], buf_v)
        for k in range(QS // 16):
            s = pl.ds(k * 16, 16)
            acc_v[s] = acc_v[s] + buf_v[s]
        return c
    lax.fori_loop(1, NW, merge, 0)

    one = jnp.ones((16,), jnp.float32)
    for k in range(QS // 16):
        s = pl.ds(k * 16, 16)
        acc_v[s] = one / acc_v[s]
    pltpu.sync_copy(acc_v, out_hbm.at[pl.ds(off, QS)])


def _merge_rdenom(parts):
    k = functools.partial(
        pl.kernel,
        mesh=_mesh(),
        compiler_params=_SC_PARAMS,
        out_type=jax.ShapeDtypeStruct((QP,), jnp.float32),
        scratch_types=[
            pltpu.VMEM((QS,), jnp.float32),
            pltpu.VMEM((QS,), jnp.float32),
        ],
    )(_k4_body)
    return k(parts)


def _k5_body(sv_hbm, idx_hbm, exps_hbm, rden_hbm, scores_hbm, partial_hbm,
             sva_v, ea_v, ia_v, sca_v, svb_v, eb_v, ib_v, scb_v,
             w_v, rden_v, accum_s,
             sa_sv, sa_e, sa_i, sb_sv, sb_e, sb_i, s_sc, s_w0, s_w1):
    cid = lax.axis_index("c")
    sid = lax.axis_index("s")
    wid = sid * NC + cid
    base = wid * EPW
    pltpu.sync_copy(rden_hbm, rden_v)

    zero = jnp.zeros((16,), jnp.float32)

    def zrow(r, c):
        for k in range(8):
            w_v[r, pl.ds(k * 16, 16)] = zero
        return c
    lax.fori_loop(0, CH, zrow, 0)
    for z in range(QT // CH):
        pltpu.sync_copy(w_v, accum_s.at[pl.ds(sid * QT + z * CH, CH), :])
    plsc.subcore_barrier()

    def start(j, sv_v, e_v, i_v, s_sv, s_e, s_i):
        cs = pl.ds(base + j * CH, CH)
        pltpu.make_async_copy(sv_hbm.at[cs], sv_v, s_sv).start()
        pltpu.make_async_copy(exps_hbm.at[cs], e_v, s_e).start()
        pltpu.make_async_copy(idx_hbm.at[cs], i_v, s_i).start()

    def wait(j, sv_v, e_v, i_v, s_sv, s_e, s_i):
        cs = pl.ds(base + j * CH, CH)
        pltpu.make_async_copy(sv_hbm.at[cs], sv_v, s_sv).wait()
        pltpu.make_async_copy(exps_hbm.at[cs], e_v, s_e).wait()
        pltpu.make_async_copy(idx_hbm.at[cs], i_v, s_i).wait()

    HH = CH // 2

    def compute(j, sv_v, e_v, i_v, sc_v, s_sc, s_w0, s_w1):
        for g in range(NG):
            s = pl.ds(g * 16, 16)
            i16 = i_v[s]
            r16 = plsc.load_gather(rden_v, [i16])
            sc_v[s] = e_v[s] * r16

        dsc = pltpu.async_copy(sc_v, scores_hbm.at[pl.ds(base + j * CH, CH)],
                               s_sc)

        dws = []
        for h, s_w in ((0, s_w0), (1, s_w1)):
            def row(r, cc, h=h):
                splat = plsc.load_gather(
                    sc_v, [jnp.full((16,), h * HH, jnp.int32) + r])
                for k in range(8):
                    sl = pl.ds(k * 16, 16)
                    w_v[h * HH + r, sl] = sv_v[h * HH + r, sl] * splat
                return cc
            lax.fori_loop(0, HH, row, 0)
            dws.append(pltpu.async_copy(
                w_v.at[pl.ds(h * HH, HH), :],
                accum_s.at[i_v.at[pl.ds(h * HH, HH)]], s_w, add=True))
        dsc.wait()
        for d in dws:
            d.wait()

    start(0, sva_v, ea_v, ia_v, sa_sv, sa_e, sa_i)

    def pair(i, c):
        j = 2 * i
        start(j + 1, svb_v, eb_v, ib_v, sb_sv, sb_e, sb_i)
        wait(j, sva_v, ea_v, ia_v, sa_sv, sa_e, sa_i)
        compute(j, sva_v, ea_v, ia_v, sca_v, s_sc, s_w0, s_w1)
        start(j + 2, sva_v, ea_v, ia_v, sa_sv, sa_e, sa_i)
        wait(j + 1, svb_v, eb_v, ib_v, sb_sv, sb_e, sb_i)
        compute(j + 1, svb_v, eb_v, ib_v, scb_v, s_sc, s_w0, s_w1)
        return c
    lax.fori_loop(0, (NCH - 1) // 2, pair, 0)
    wait(NCH - 1, sva_v, ea_v, ia_v, sa_sv, sa_e, sa_i)
    compute(NCH - 1, sva_v, ea_v, ia_v, sca_v, s_sc, s_w0, s_w1)

    plsc.subcore_barrier()
    pltpu.sync_copy(accum_s.at[pl.ds(sid * QT, QT), :],
                    partial_hbm.at[cid, pl.ds(sid * QT, QT), :])


def _pass3(sv, idxf, exps, rdenom):
    k = functools.partial(
        pl.kernel,
        mesh=_mesh(),
        compiler_params=_SC_PARAMS,
        out_type=(
            jax.ShapeDtypeStruct((E,), jnp.float32),
            jax.ShapeDtypeStruct((NC, QP, D), jnp.float32),
        ),
        scratch_types=[
            pltpu.VMEM((CH, D), jnp.float32),
            pltpu.VMEM((CH,), jnp.float32),
            pltpu.VMEM((CH,), jnp.int32),
            pltpu.VMEM((CH,), jnp.float32),
            pltpu.VMEM((CH, D), jnp.float32),
            pltpu.VMEM((CH,), jnp.float32),
            pltpu.VMEM((CH,), jnp.int32),
            pltpu.VMEM((CH,), jnp.float32),
            pltpu.VMEM((CH, D), jnp.float32),
            pltpu.VMEM((QP,), jnp.float32),
            pltpu.VMEM_SHARED((QP, D), jnp.float32),
            pltpu.SemaphoreType.DMA,
            pltpu.SemaphoreType.DMA,
            pltpu.SemaphoreType.DMA,
            pltpu.SemaphoreType.DMA,
            pltpu.SemaphoreType.DMA,
            pltpu.SemaphoreType.DMA,
            pltpu.SemaphoreType.DMA,
            pltpu.SemaphoreType.DMA,
            pltpu.SemaphoreType.DMA,
        ],
    )(_k5_body)
    return k(sv, idxf, exps, rdenom)


def _k6_body(p_ref, o_ref):
    o_ref[...] = p_ref[0] + p_ref[1]


def _merge_partials(partial):
    return pl.pallas_call(
        _k6_body,
        grid=(25,),
        in_specs=[pl.BlockSpec((2, 400, D), lambda i: (0, i, 0))],
        out_specs=pl.BlockSpec((400, D), lambda i: (i, 0)),
        out_shape=jax.ShapeDtypeStruct((Q, D), jnp.float32),
    )(partial)


def kernel(scattered_values, indices, queries, Wq, bq, Wk, bk):
    del bk
    sv = scattered_values
    idxf = indices.astype(jnp.int32)
    t = _project_t(queries, Wq, bq, Wk)
    probs, segmax_part = _pass1(sv, idxf, t)
    segmax = _merge_max(segmax_part)
    exps, denom_part = _pass2(probs, idxf, segmax)
    rdenom = _merge_rdenom(denom_part)
    scores, partial = _pass3(sv, idxf, exps, rdenom)
    attn = _merge_partials(partial)
    return scores, attn
